# Initial kernel scaffold; baseline (speedup 1.0000x reference)
import jax
import jax.numpy as jnp
from jax import lax
from jax.experimental import pallas as pl
from jax.experimental.pallas import tpu as pltpu
from jax.experimental.pallas import tpu_sc as plsc

B = 4096
L = 200
D = 32
LANES = 16

NC = 2
NS = 16
NW = NC * NS
SEQ_PER_W = B // NW
S_CHUNK = 8
CHUNK_ROWS = S_CHUNK * L
N_CHUNKS = SEQ_PER_W // S_CHUNK


def _emb_body(table_hbm, idx_hbm, pe_hbm, scale_hbm, out_hbm,
              idx_v, rows_v, pe_v, scale_v, sem):
    wid = lax.axis_index("s") * NC + lax.axis_index("c")

    pltpu.sync_copy(pe_hbm.at[pl.ds(0, L)], pe_v)
    pltpu.sync_copy(scale_hbm, scale_v)
    sv = scale_v[...]

    base_row = wid * SEQ_PER_W * L

    def chunk_body(c, carry):
        row0 = base_row + c * CHUNK_ROWS
        pltpu.sync_copy(idx_hbm.at[pl.ds(row0, CHUNK_ROWS)], idx_v)
        pltpu.async_copy(table_hbm.at[idx_v], rows_v, sem).wait()

        def l_body(l, carry2):
            pe_lo = pe_v[l, pl.ds(0, LANES)]
            pe_hi = pe_v[l, pl.ds(LANES, LANES)]
            for s in range(S_CHUNK):
                r = s * L + l
                rows_v[r, pl.ds(0, LANES)] = (
                    rows_v[r, pl.ds(0, LANES)] * sv + pe_lo)
                rows_v[r, pl.ds(LANES, LANES)] = (
                    rows_v[r, pl.ds(LANES, LANES)] * sv + pe_hi)
            return carry2

        lax.fori_loop(0, L, l_body, 0)
        pltpu.sync_copy(rows_v, out_hbm.at[pl.ds(row0, CHUNK_ROWS)])
        return carry

    lax.fori_loop(0, N_CHUNKS, chunk_body, 0)


def kernel(x, table, pe, scale):
    idx = x.reshape(-1).astype(jnp.int32)
    scale_v = jnp.broadcast_to(scale.astype(jnp.float32), (LANES,))
    mesh = plsc.VectorSubcoreMesh(core_axis_name="c", subcore_axis_name="s")
    out = pl.kernel(
        _emb_body,
        out_type=jax.ShapeDtypeStruct((B * L, D), jnp.float32),
        mesh=mesh,
        scratch_types=[
            pltpu.VMEM((CHUNK_ROWS,), jnp.int32),
            pltpu.VMEM((CHUNK_ROWS, D), jnp.float32),
            pltpu.VMEM((L, D), jnp.float32),
            pltpu.VMEM((LANES,), jnp.float32),
            pltpu.SemaphoreType.DMA,
        ],
    )(table, idx, pe, scale_v)
    return out.reshape(B, L, D)


# speedup vs baseline: 1.4266x; 1.4266x over previous
import jax
import jax.numpy as jnp
from jax import lax
from jax.experimental import pallas as pl
from jax.experimental.pallas import tpu as pltpu
from jax.experimental.pallas import tpu_sc as plsc

B = 4096
L = 200
D = 32
LANES = 16

NC = 2
NS = 16
NW = NC * NS
SEQ_PER_W = B // NW
S_CHUNK = 8
CHUNK_ROWS = S_CHUNK * L
N_CHUNKS = SEQ_PER_W // S_CHUNK


def _emb_body(table_hbm, idx_hbm, pe_hbm, scale_hbm, out_hbm,
              idx_v, rows_v, pe_v, scale_v, sem):
    wid = lax.axis_index("s") * NC + lax.axis_index("c")

    pltpu.sync_copy(pe_hbm.at[pl.ds(0, L)], pe_v)
    pltpu.sync_copy(scale_hbm, scale_v)
    sv = scale_v[...]

    base_row = wid * SEQ_PER_W * L

    def chunk_body(c, carry):
        row0 = base_row + c * CHUNK_ROWS
        pltpu.sync_copy(idx_hbm.at[pl.ds(row0, CHUNK_ROWS)], idx_v)
        pltpu.async_copy(table_hbm.at[idx_v], rows_v, sem).wait()

        def l_body(l, carry2):
            pe_lo = pe_v[l, pl.ds(0, LANES)]
            pe_hi = pe_v[l, pl.ds(LANES, LANES)]
            for s in range(S_CHUNK):
                r = s * L + l
                rows_v[r, pl.ds(0, LANES)] = (
                    rows_v[r, pl.ds(0, LANES)] * sv + pe_lo)
                rows_v[r, pl.ds(LANES, LANES)] = (
                    rows_v[r, pl.ds(LANES, LANES)] * sv + pe_hi)
            return carry2

        lax.fori_loop(0, L, l_body, 0)
        pltpu.sync_copy(rows_v, out_hbm.at[pl.ds(row0, CHUNK_ROWS)])
        return carry

    lax.fori_loop(0, N_CHUNKS, chunk_body, 0)


def kernel(x, table, pe, scale):
    idx = x.reshape(-1).astype(jnp.int32)
    scale_v = jnp.broadcast_to(scale.astype(jnp.float32), (LANES,))
    mesh = plsc.VectorSubcoreMesh(core_axis_name="c", subcore_axis_name="s")
    out = pl.kernel(
        _emb_body,
        out_type=jax.ShapeDtypeStruct((B * L, D), jnp.float32),
        mesh=mesh,
        compiler_params=pltpu.CompilerParams(use_tc_tiling_on_sc=False),
        scratch_types=[
            pltpu.VMEM((CHUNK_ROWS,), jnp.int32),
            pltpu.VMEM((CHUNK_ROWS, D), jnp.float32),
            pltpu.VMEM((L, D), jnp.float32),
            pltpu.VMEM((LANES,), jnp.float32),
            pltpu.SemaphoreType.DMA,
        ],
    )(table, idx, pe, scale_v)
    return out.reshape(B, L, D)
